# baseline (speedup 1.0000x reference)
import functools

import jax
import jax.numpy as jnp
from jax import lax
from jax.experimental import pallas as pl
from jax.experimental.pallas import tpu as pltpu
from jax.experimental.pallas import tpu_sc as plsc

NC = 2
NS = 16
NW = NC * NS
CH = 128
LANES = 16


@functools.cache
def _build(batch: int, seq: int, dim: int):
    mesh = plsc.VectorSubcoreMesh(
        core_axis_name="c", subcore_axis_name="s", num_cores=NC,
        num_subcores=NS)
    rows_per_w = batch // NW
    ch_per_row = (seq + CH - 1) // CH
    n_lrow = rows_per_w * ch_per_row
    pair = 2 * dim
    n_out = batch * seq + NW

    @functools.partial(
        pl.kernel,
        out_type=jax.ShapeDtypeStruct((n_out, pair), jnp.float32),
        mesh=mesh,
        scratch_types=[
            pltpu.VMEM((rows_per_w, seq), jnp.int32),
            pltpu.VMEM((n_lrow, CH), jnp.int32),
            pltpu.VMEM((n_lrow, CH), jnp.int32),
            pltpu.VMEM((n_lrow, CH), jnp.int32),
            pltpu.VMEM((CH, pair), jnp.float32),
            pltpu.VMEM((CH, pair), jnp.float32),
            pltpu.SemaphoreType.DMA,
            pltpu.SemaphoreType.DMA,
        ],
        compiler_params=pltpu.CompilerParams(
            use_tc_tiling_on_sc=True, needs_layout_passes=False),
    )
    def embed(x_hbm, tbl_hbm, out_hbm, idx_v, pv, qe, qo, ebuf, obuf,
              gsem, ssem):
        wid = lax.axis_index("s") * NC + lax.axis_index("c")
        row0 = wid * rows_per_w
        pltpu.sync_copy(x_hbm.at[pl.ds(row0, rows_per_w)], idx_v)

        iot = lax.iota(jnp.int32, LANES)
        sacrifice = jnp.int32(batch * seq) + wid + iot * 0
        zeros = iot * 0

        def prep_row(r, carry):
            rvec = r + zeros
            for c in range(ch_per_row):
                base = c * CH
                width = min(CH, seq - base)
                lr = r * ch_per_row + c
                for g in range(CH // LANES):
                    lcol = g * LANES
                    if lcol >= width:
                        pv[lr, pl.ds(lcol, LANES)] = zeros
                        qe[lr, pl.ds(lcol, LANES)] = sacrifice
                        qo[lr, pl.ds(lcol, LANES)] = sacrifice
                        continue
                    tcol = base + lcol
                    if lcol + LANES <= width:
                        iv = idx_v[r, pl.ds(tcol, LANES)]
                        valid = None
                    else:
                        iv = plsc.load_gather(
                            idx_v,
                            [rvec, jnp.minimum(tcol + iot, seq - 1)])
                        valid = (lcol + iot) < width
                    pos = (row0 + r) * seq + tcol + iot
                    even = (iv & 1) == 0
                    if valid is not None:
                        even = valid & even
                        odd = valid & ~((iv & 1) == 0)
                        pval = jnp.where(valid,
                                         lax.shift_right_logical(iv, 1),
                                         zeros)
                    else:
                        odd = ~even
                        pval = lax.shift_right_logical(iv, 1)
                    pv[lr, pl.ds(lcol, LANES)] = pval
                    qe[lr, pl.ds(lcol, LANES)] = (
                        jnp.where(even, pos, sacrifice))
                    qo[lr, pl.ds(lcol, LANES)] = (
                        jnp.where(odd, pos, sacrifice))
            return carry

        lax.fori_loop(0, rows_per_w, prep_row, 0)

        def do_chunk(lr, carry):
            pltpu.async_copy(tbl_hbm.at[pv.at[lr]], ebuf, gsem).wait()

            def shift(t, cc):
                for k in range(dim // LANES):
                    obuf[t, pl.ds(k * LANES, LANES)] = (
                        ebuf[t, pl.ds(dim + k * LANES, LANES)])
                return cc

            lax.fori_loop(0, CH, shift, 0)

            edst = out_hbm.at[qe.at[lr]]
            odst = out_hbm.at[qo.at[lr]]
            pltpu.async_copy(ebuf, edst, ssem)
            pltpu.async_copy(obuf, odst, ssem)
            pltpu.make_async_copy(ebuf, edst, ssem).wait()
            pltpu.make_async_copy(obuf, odst, ssem).wait()
            return carry

        lax.fori_loop(0, n_lrow, do_chunk, 0)

    return embed


def kernel(x, input_embedding):
    batch, seq = x.shape
    vocab, dim = input_embedding.shape
    tbl = input_embedding.reshape(vocab // 2, 2 * dim)
    out = _build(batch, seq, dim)(x, tbl)
    return out[:batch * seq, :dim].reshape(batch, seq, dim)


# speedup vs baseline: 1.0015x; 1.0015x over previous
import functools

import jax
import jax.numpy as jnp
from jax import lax
from jax.experimental import pallas as pl
from jax.experimental.pallas import tpu as pltpu
from jax.experimental.pallas import tpu_sc as plsc

NC = 2
NS = 16
NW = NC * NS
CH = 128
LANES = 16


@functools.cache
def _build(batch: int, seq: int, dim: int):
    mesh = plsc.VectorSubcoreMesh(
        core_axis_name="c", subcore_axis_name="s", num_cores=NC,
        num_subcores=NS)
    rows_per_w = batch // NW
    ch_per_row = (seq + CH - 1) // CH
    n_lrow = rows_per_w * ch_per_row
    pair = 2 * dim
    n_out = batch * seq + NW
    seq_pad = ((seq + LANES - 1) // LANES) * LANES
    ch_per_row = (seq_pad + CH - 1) // CH
    n_lrow = rows_per_w * ch_per_row

    @functools.partial(
        pl.kernel,
        out_type=jax.ShapeDtypeStruct((n_out, pair), jnp.float32),
        mesh=mesh,
        scratch_types=[
            pltpu.VMEM((rows_per_w, seq_pad), jnp.int32),
            pltpu.VMEM((n_lrow, CH), jnp.int32),
            pltpu.VMEM((n_lrow, CH), jnp.int32),
            pltpu.VMEM((n_lrow, CH), jnp.int32),
            pltpu.VMEM((CH, pair), jnp.float32),
            pltpu.VMEM((CH, pair), jnp.float32),
            pltpu.SemaphoreType.DMA,
            pltpu.SemaphoreType.DMA,
        ],
        compiler_params=pltpu.CompilerParams(use_tc_tiling_on_sc=True),
    )
    def embed(x_hbm, tbl_hbm, out_hbm, idx_v, pv, qe, qo, ebuf, obuf,
              gsem, ssem):
        wid = lax.axis_index("s") * NC + lax.axis_index("c")
        row0 = wid * rows_per_w
        pltpu.sync_copy(x_hbm.at[pl.ds(row0, rows_per_w)], idx_v)

        iot = lax.iota(jnp.int32, LANES)
        sacrifice = jnp.int32(batch * seq) + wid + iot * 0
        zeros = iot * 0

        def prep_row(r, carry):
            rvec = r + zeros
            for c in range(ch_per_row):
                base = c * CH
                lr = r * ch_per_row + c
                for g in range(CH // LANES):
                    lcol = g * LANES
                    tcol = base + lcol
                    if tcol >= seq_pad:
                        pv[lr, pl.ds(lcol, LANES)] = zeros
                        qe[lr, pl.ds(lcol, LANES)] = sacrifice
                        qo[lr, pl.ds(lcol, LANES)] = sacrifice
                        continue
                    iv = idx_v[r, pl.ds(tcol, LANES)]
                    pos = (row0 + r) * seq + tcol + iot
                    if tcol + LANES > seq:
                        vi = jnp.minimum(
                            jnp.maximum(jnp.int32(seq - tcol) - iot, 0), 1)
                        iv = iv * vi
                        pos = pos * vi + sacrifice * (1 - vi)
                    even = (iv & 1) == 0
                    pv[lr, pl.ds(lcol, LANES)] = (
                        lax.shift_right_logical(iv, 1))
                    qe[lr, pl.ds(lcol, LANES)] = (
                        jnp.where(even, pos, sacrifice))
                    qo[lr, pl.ds(lcol, LANES)] = (
                        jnp.where(even, sacrifice, pos))
            return carry

        lax.fori_loop(0, rows_per_w, prep_row, 0)

        def do_chunk(lr, carry):
            pltpu.async_copy(tbl_hbm.at[pv.at[lr]], ebuf, gsem).wait()

            def shift(t, cc):
                for k in range(dim // LANES):
                    obuf[t, pl.ds(k * LANES, LANES)] = (
                        ebuf[t, pl.ds(dim + k * LANES, LANES)])
                return cc

            lax.fori_loop(0, CH, shift, 0)

            edst = out_hbm.at[qe.at[lr]]
            odst = out_hbm.at[qo.at[lr]]
            pltpu.async_copy(ebuf, edst, ssem)
            pltpu.async_copy(obuf, odst, ssem)
            pltpu.make_async_copy(ebuf, edst, ssem).wait()
            pltpu.make_async_copy(obuf, odst, ssem).wait()
            return carry

        lax.fori_loop(0, n_lrow, do_chunk, 0)

    return embed


def kernel(x, input_embedding):
    batch, seq = x.shape
    vocab, dim = input_embedding.shape
    tbl = input_embedding.reshape(vocab // 2, 2 * dim)
    seq_pad = ((seq + 15) // 16) * 16
    xp = jnp.pad(x, ((0, 0), (0, seq_pad - seq))) if seq_pad != seq else x
    out = _build(batch, seq, dim)(xp, tbl)
    return out[:batch * seq, :dim].reshape(batch, seq, dim)


# speedup vs baseline: 5.5785x; 5.5704x over previous
import functools

import jax
import jax.numpy as jnp
from jax import lax
from jax.experimental import pallas as pl
from jax.experimental.pallas import tpu as pltpu
from jax.experimental.pallas import tpu_sc as plsc

NC = 2
NS = 16
NW = NC * NS
GCH = 128
NBUF = 4
LOOKAHEAD = 2


@functools.cache
def _build(batch: int, seq: int, dim: int):
    mesh = plsc.VectorSubcoreMesh(core_axis_name="c", subcore_axis_name="s")
    rows_per_w = batch // NW
    splits = []
    off = 0
    while off < seq:
        splits.append((off, min(GCH, seq - off)))
        off += GCH

    @functools.partial(
        pl.kernel,
        out_type=jax.ShapeDtypeStruct((batch, seq, dim), jnp.float32),
        mesh=mesh,
        scratch_types=[
            pltpu.VMEM((rows_per_w, seq), jnp.int32),
            pltpu.VMEM((NBUF, seq, dim), jnp.float32),
            pltpu.SemaphoreType.DMA((NBUF,)),
            pltpu.SemaphoreType.DMA((NBUF,)),
        ],
        compiler_params=pltpu.CompilerParams(use_tc_tiling_on_sc=False),
    )
    def embed(x_hbm, table_hbm, out_hbm, idx_v, rows, gsem, ssem):
        wid = lax.axis_index("s") * NC + lax.axis_index("c")
        row0 = wid * rows_per_w
        pltpu.sync_copy(x_hbm.at[pl.ds(row0, rows_per_w)], idx_v)

        def gather_start(r, b):
            for off, n in splits:
                pltpu.async_copy(
                    table_hbm.at[idx_v.at[r, pl.ds(off, n)]],
                    rows.at[b, pl.ds(off, n)], gsem.at[b])

        def gather_wait(r, b):
            for off, n in splits:
                pltpu.make_async_copy(
                    table_hbm.at[idx_v.at[r, pl.ds(off, n)]],
                    rows.at[b, pl.ds(off, n)], gsem.at[b]).wait()

        def store_start(r, b):
            pltpu.async_copy(rows.at[b], out_hbm.at[row0 + r], ssem.at[b])

        def store_wait(b):
            pltpu.make_async_copy(rows.at[b], out_hbm.at[row0], ssem.at[b]).wait()

        for r in range(LOOKAHEAD):
            gather_start(r, r % NBUF)

        def step(r, carry):
            r_pre = r + LOOKAHEAD

            @pl.when(r_pre < rows_per_w)
            def _():
                b_pre = lax.rem(r_pre, NBUF)

                @pl.when(r_pre >= NBUF)
                def _():
                    store_wait(b_pre)

                gather_start(r_pre, b_pre)

            b = lax.rem(r, NBUF)
            gather_wait(r, b)
            store_start(r, b)
            return carry

        lax.fori_loop(0, rows_per_w, step, 0)
        for b in range(min(NBUF, rows_per_w)):
            store_wait(b)

    return embed


def kernel(x, input_embedding):
    batch, seq = x.shape
    dim = input_embedding.shape[1]
    return _build(batch, seq, dim)(x, input_embedding)
